# baseline (speedup 1.0000x reference)
import functools

import jax
import jax.numpy as jnp
from jax import lax
from jax.experimental import pallas as pl
from jax.experimental.pallas import tpu as pltpu
from jax.experimental.pallas import tpu_sc as plsc

NC = 2
NS = 16
NW = NC * NS


def _sc_edge_segment_sum(values, src, dst, num_segments, chunk=128):
    n_rows, d = values.shape
    n_edges = dst.shape[0]
    block = chunk * NW
    cpw = -(-n_edges // block)
    cpw = -(-cpw // 16) * 16
    e_pad = cpw * block
    pad = e_pad - n_edges
    n_dummy = 512
    if pad:
        src = jnp.concatenate([src, jnp.zeros((pad,), jnp.int32)])
        dst = jnp.concatenate(
            [dst, num_segments + (jnp.arange(pad, dtype=jnp.int32) % n_dummy)])
    src3 = src.reshape(NW, cpw, chunk)
    dst3 = dst.reshape(NW, cpw, chunk)
    seg_pad = num_segments + n_dummy
    rpt = (num_segments // NS) // 8 * 8
    tail = num_segments - rpt * NS
    zeros = jnp.zeros((seg_pad, d), jnp.float32)
    mesh = plsc.VectorSubcoreMesh(core_axis_name="c", subcore_axis_name="s")

    hpw = cpw // 2
    scratch = [
        pltpu.VMEM((hpw, chunk), jnp.int32),
        pltpu.VMEM((hpw, chunk), jnp.int32),
        pltpu.VMEM((2, chunk, d), jnp.float32),
        pltpu.VMEM_SHARED((seg_pad, d), jnp.float32),
        pltpu.SemaphoreType.DMA,
        pltpu.SemaphoreType.DMA,
    ]

    def body(val_hbm, src_hbm, dst_hbm, zero_hbm, out_hbm,
             src_v, dst_v, rows_v, acc, g0, g1):
        cid = lax.axis_index("c")
        sid = lax.axis_index("s")
        wid = sid * NC + cid
        gsem = (g0, g1)

        pltpu.sync_copy(zero_hbm.at[pl.ds(sid * rpt, rpt)],
                        acc.at[pl.ds(sid * rpt, rpt)])
        rest = seg_pad - rpt * NS
        @pl.when(sid == 0)
        def _():
            pltpu.sync_copy(zero_hbm.at[pl.ds(rpt * NS, rest)],
                            acc.at[pl.ds(rpt * NS, rest)])
        plsc.subcore_barrier()

        def run_half(h):
            base = h * hpw
            pltpu.sync_copy(src_hbm.at[wid, pl.ds(base, hpw)], src_v)
            pltpu.sync_copy(dst_hbm.at[wid, pl.ds(base, hpw)], dst_v)
            def step(c, carry):
                pltpu.async_copy(val_hbm.at[src_v.at[c]],
                                 rows_v.at[0], g0).wait()
                pltpu.sync_copy(rows_v.at[0], acc.at[dst_v.at[c]], add=True)
                return carry

            lax.fori_loop(0, hpw, step, 0)

        run_half(0)
        run_half(1)
        plsc.subcore_barrier()
        pltpu.sync_copy(acc.at[pl.ds(sid * rpt, rpt)],
                        out_hbm.at[cid, pl.ds(sid * rpt, rpt)])
        if tail:
            @pl.when(sid == 0)
            def _():
                pltpu.sync_copy(acc.at[pl.ds(rpt * NS, tail)],
                                out_hbm.at[cid, pl.ds(rpt * NS, tail)])

    return pl.kernel(
        body,
        out_type=jax.ShapeDtypeStruct((NC, num_segments, d), jnp.float32),
        mesh=mesh,
        scratch_types=scratch,
    )(values, src3, dst3, zeros)


def _sc_pool_segment_sum(values, dst, num_segments, chunk=80):
    n_rows, d = values.shape
    n_chunks = n_rows // chunk
    rpt = (num_segments // NS) // 8 * 8
    tail = num_segments - rpt * NS
    zeros = jnp.zeros((num_segments, d), jnp.float32)
    mesh = plsc.VectorSubcoreMesh(core_axis_name="c", subcore_axis_name="s")

    scratch = [
        pltpu.VMEM((1, chunk), jnp.int32),
        pltpu.VMEM((chunk, d), jnp.float32),
        pltpu.VMEM_SHARED((num_segments, d), jnp.float32),
        pltpu.SemaphoreType.DMA,
    ]

    def body(val_hbm, dst_hbm, zero_hbm, out_hbm, dst_v, rows_v, acc, sem):
        cid = lax.axis_index("c")
        sid = lax.axis_index("s")
        wid = sid * NC + cid

        pltpu.sync_copy(zero_hbm.at[pl.ds(sid * rpt, rpt)],
                        acc.at[pl.ds(sid * rpt, rpt)])
        if tail:
            @pl.when(sid == 0)
            def _():
                pltpu.sync_copy(zero_hbm.at[pl.ds(rpt * NS, tail)],
                                acc.at[pl.ds(rpt * NS, tail)])
        plsc.subcore_barrier()

        n_my = (n_chunks - wid + NW - 1) // NW

        def step(i, carry):
            c = wid + i * NW
            off = c * chunk
            pltpu.sync_copy(dst_hbm.at[pl.ds(off, chunk)], dst_v.at[0])
            pltpu.sync_copy(val_hbm.at[pl.ds(off, chunk)], rows_v)
            pltpu.sync_copy(rows_v, acc.at[dst_v.at[0]], add=True)
            return carry

        lax.fori_loop(0, n_my, step, 0)
        plsc.subcore_barrier()
        pltpu.sync_copy(acc.at[pl.ds(sid * rpt, rpt)],
                        out_hbm.at[cid, pl.ds(sid * rpt, rpt)])
        if tail:
            @pl.when(sid == 0)
            def _():
                pltpu.sync_copy(acc.at[pl.ds(rpt * NS, tail)],
                                out_hbm.at[cid, pl.ds(rpt * NS, tail)])

    return pl.kernel(
        body,
        out_type=jax.ShapeDtypeStruct((NC, num_segments, d), jnp.float32),
        mesh=mesh,
        scratch_types=scratch,
    )(values, dst, zeros)


def _tc_layer(h, parts, eps, w1, b1, w2, b2, gamma, beta):
    n, d = h.shape

    def body(eps_ref, h_ref, p_ref, w1_ref, b1_ref, w2_ref, b2_ref,
             g_ref, be_ref, o_ref):
        hv = h_ref[...]
        z = hv + eps_ref[0] * hv + p_ref[0] + p_ref[1]
        a = jnp.dot(z, w1_ref[...], preferred_element_type=jnp.float32,
                    precision=lax.Precision.HIGHEST) + b1_ref[...]
        a = jnp.maximum(a, 0.0)
        z2 = jnp.dot(a, w2_ref[...], preferred_element_type=jnp.float32,
                     precision=lax.Precision.HIGHEST) + b2_ref[...]
        mu = jnp.mean(z2, axis=0, keepdims=True)
        var = jnp.mean(z2 * z2, axis=0, keepdims=True) - mu * mu
        zn = (z2 - mu) * lax.rsqrt(var + 1e-5) * g_ref[...] + be_ref[...]
        o_ref[...] = jnp.maximum(zn, 0.0) + hv

    smem = pl.BlockSpec(memory_space=pltpu.SMEM)
    vmem = pl.BlockSpec(memory_space=pltpu.VMEM)
    return pl.pallas_call(
        body,
        out_shape=jax.ShapeDtypeStruct((n, d), jnp.float32),
        in_specs=[smem] + [vmem] * 8,
        out_specs=vmem,
    )(jnp.reshape(eps, (1,)), h, parts, w1, jnp.reshape(b1, (1, d)), w2,
      jnp.reshape(b2, (1, d)), jnp.reshape(gamma, (1, d)),
      jnp.reshape(beta, (1, d)))


def _tc_head(parts, w1, b1, w2, b2):
    g, d = parts.shape[1], parts.shape[2]
    d_out = w2.shape[1]

    def body(p_ref, w1_ref, b1_ref, w2_ref, b2_ref, o_ref):
        zp = p_ref[0] + p_ref[1]
        a = jnp.dot(zp, w1_ref[...], preferred_element_type=jnp.float32,
                    precision=lax.Precision.HIGHEST) + b1_ref[...]
        a = jnp.maximum(a, 0.0)
        o_ref[...] = jnp.dot(a, w2_ref[...], preferred_element_type=jnp.float32,
                             precision=lax.Precision.HIGHEST) + b2_ref[...]

    vmem = pl.BlockSpec(memory_space=pltpu.VMEM)
    return pl.pallas_call(
        body,
        out_shape=jax.ShapeDtypeStruct((g, d_out), jnp.float32),
        in_specs=[vmem] * 5,
        out_specs=vmem,
    )(parts, w1, jnp.reshape(b1, (1, d)), w2, jnp.reshape(b2, (1, d_out)))


def kernel(x, edge_index, batch, l0_eps, l0_W1, l0_b1, l0_W2, l0_b2, l0_gamma,
           l0_beta, l1_eps, l1_W1, l1_b1, l1_W2, l1_b2, l1_gamma, l1_beta,
           l2_eps, l2_W1, l2_b1, l2_W2, l2_b2, l2_gamma, l2_beta, head_W1,
           head_b1, head_W2, head_b2):
    src = edge_index[0]
    dst = edge_index[1]
    n = x.shape[0]
    g = 512

    layers = [
        (l0_eps, l0_W1, l0_b1, l0_W2, l0_b2, l0_gamma, l0_beta),
        (l1_eps, l1_W1, l1_b1, l1_W2, l1_b2, l1_gamma, l1_beta),
        (l2_eps, l2_W1, l2_b1, l2_W2, l2_b2, l2_gamma, l2_beta),
    ]
    h = x
    for (eps, w1, b1, w2, b2, gamma, beta) in layers:
        parts = _sc_edge_segment_sum(h, src, dst, n, chunk=128)
        h = _tc_layer(h, parts, eps, w1, b1, w2, b2, gamma, beta)

    pool_parts = _sc_pool_segment_sum(h, batch, g, chunk=80)
    return _tc_head(pool_parts, head_W1, head_b1, head_W2, head_b2)


# speedup vs baseline: 1.1037x; 1.1037x over previous
import functools

import jax
import jax.numpy as jnp
from jax import lax
from jax.experimental import pallas as pl
from jax.experimental.pallas import tpu as pltpu
from jax.experimental.pallas import tpu_sc as plsc

NC = 2
NS = 16
NW = NC * NS


def _sc_edge_segment_sum(values, src, dst, num_segments, chunk=128):
    n_rows, d = values.shape
    n_edges = dst.shape[0]
    block = chunk * NW
    cpw = -(-n_edges // block)
    cpw = -(-cpw // 16) * 16
    e_pad = cpw * block
    pad = e_pad - n_edges
    n_dummy = 512
    if pad:
        src = jnp.concatenate([src, jnp.zeros((pad,), jnp.int32)])
        dst = jnp.concatenate(
            [dst, num_segments + (jnp.arange(pad, dtype=jnp.int32) % n_dummy)])
    idx4 = jnp.stack(
        [src.reshape(NW, cpw, chunk), dst.reshape(NW, cpw, chunk)], axis=2)
    seg_pad = num_segments + n_dummy
    rpt = (num_segments // NS) // 8 * 8
    tail = num_segments - rpt * NS
    zeros = jnp.zeros((seg_pad, d), jnp.float32)
    mesh = plsc.VectorSubcoreMesh(core_axis_name="c", subcore_axis_name="s")

    scratch = [
        pltpu.VMEM((2, 2, chunk), jnp.int32),
        pltpu.VMEM((2, chunk, d), jnp.float32),
        pltpu.VMEM_SHARED((seg_pad, d), jnp.float32),
        pltpu.SemaphoreType.DMA,
        pltpu.SemaphoreType.DMA,
        pltpu.SemaphoreType.DMA,
        pltpu.SemaphoreType.DMA,
    ]

    def body(val_hbm, idx_hbm, zero_hbm, out_hbm,
             idx_v, rows_v, acc, i0, i1, g0, g1):
        cid = lax.axis_index("c")
        sid = lax.axis_index("s")
        wid = sid * NC + cid
        isem = (i0, i1)
        gsem = (g0, g1)

        def idx_copy(c, b):
            return pltpu.make_async_copy(idx_hbm.at[wid, c], idx_v.at[b],
                                         isem[b])

        def gather(b):
            return pltpu.make_async_copy(val_hbm.at[idx_v.at[b, 0]],
                                         rows_v.at[b], gsem[b])

        def scatter(b):
            pltpu.sync_copy(rows_v.at[b], acc.at[idx_v.at[b, 1]], add=True)

        idx_copy(0, 0).start()
        idx_copy(1, 1).start()
        pltpu.sync_copy(zero_hbm.at[pl.ds(sid * rpt, rpt)],
                        acc.at[pl.ds(sid * rpt, rpt)])
        rest = seg_pad - rpt * NS
        @pl.when(sid == 0)
        def _():
            pltpu.sync_copy(zero_hbm.at[pl.ds(rpt * NS, rest)],
                            acc.at[pl.ds(rpt * NS, rest)])
        plsc.subcore_barrier()

        idx_copy(0, 0).wait()
        gather(0).start()

        def step(j, carry):
            for b in range(2):
                c = 2 * j + b
                nb = 1 - b
                idx_copy(c + 1, nb).wait()
                gather(nb).start()
                gather(b).wait()
                scatter(b)
                idx_copy(c + 2, b).start()
            return carry

        lax.fori_loop(0, (cpw - 2) // 2, step, 0)
        idx_copy(cpw - 1, 1).wait()
        gather(1).start()
        gather(0).wait()
        scatter(0)
        gather(1).wait()
        scatter(1)
        plsc.subcore_barrier()
        pltpu.sync_copy(acc.at[pl.ds(sid * rpt, rpt)],
                        out_hbm.at[cid, pl.ds(sid * rpt, rpt)])
        if tail:
            @pl.when(sid == 0)
            def _():
                pltpu.sync_copy(acc.at[pl.ds(rpt * NS, tail)],
                                out_hbm.at[cid, pl.ds(rpt * NS, tail)])

    return pl.kernel(
        body,
        out_type=jax.ShapeDtypeStruct((NC, num_segments, d), jnp.float32),
        mesh=mesh,
        scratch_types=scratch,
    )(values, idx4, zeros)


def _sc_pool_segment_sum(values, dst, num_segments, chunk=80):
    n_rows, d = values.shape
    n_chunks = n_rows // chunk
    rpt = (num_segments // NS) // 8 * 8
    tail = num_segments - rpt * NS
    zeros = jnp.zeros((num_segments, d), jnp.float32)
    mesh = plsc.VectorSubcoreMesh(core_axis_name="c", subcore_axis_name="s")

    scratch = [
        pltpu.VMEM((1, chunk), jnp.int32),
        pltpu.VMEM((chunk, d), jnp.float32),
        pltpu.VMEM_SHARED((num_segments, d), jnp.float32),
        pltpu.SemaphoreType.DMA,
    ]

    def body(val_hbm, dst_hbm, zero_hbm, out_hbm, dst_v, rows_v, acc, sem):
        cid = lax.axis_index("c")
        sid = lax.axis_index("s")
        wid = sid * NC + cid

        pltpu.sync_copy(zero_hbm.at[pl.ds(sid * rpt, rpt)],
                        acc.at[pl.ds(sid * rpt, rpt)])
        if tail:
            @pl.when(sid == 0)
            def _():
                pltpu.sync_copy(zero_hbm.at[pl.ds(rpt * NS, tail)],
                                acc.at[pl.ds(rpt * NS, tail)])
        plsc.subcore_barrier()

        n_my = (n_chunks - wid + NW - 1) // NW

        def step(i, carry):
            c = wid + i * NW
            off = c * chunk
            pltpu.sync_copy(dst_hbm.at[pl.ds(off, chunk)], dst_v.at[0])
            pltpu.sync_copy(val_hbm.at[pl.ds(off, chunk)], rows_v)
            pltpu.sync_copy(rows_v, acc.at[dst_v.at[0]], add=True)
            return carry

        lax.fori_loop(0, n_my, step, 0)
        plsc.subcore_barrier()
        pltpu.sync_copy(acc.at[pl.ds(sid * rpt, rpt)],
                        out_hbm.at[cid, pl.ds(sid * rpt, rpt)])
        if tail:
            @pl.when(sid == 0)
            def _():
                pltpu.sync_copy(acc.at[pl.ds(rpt * NS, tail)],
                                out_hbm.at[cid, pl.ds(rpt * NS, tail)])

    return pl.kernel(
        body,
        out_type=jax.ShapeDtypeStruct((NC, num_segments, d), jnp.float32),
        mesh=mesh,
        scratch_types=scratch,
    )(values, dst, zeros)


def _tc_layer(h, parts, eps, w1, b1, w2, b2, gamma, beta):
    n, d = h.shape

    def body(eps_ref, h_ref, p_ref, w1_ref, b1_ref, w2_ref, b2_ref,
             g_ref, be_ref, o_ref):
        hv = h_ref[...]
        z = hv + eps_ref[0] * hv + p_ref[0] + p_ref[1]
        a = jnp.dot(z, w1_ref[...], preferred_element_type=jnp.float32,
                    precision=lax.Precision.HIGHEST) + b1_ref[...]
        a = jnp.maximum(a, 0.0)
        z2 = jnp.dot(a, w2_ref[...], preferred_element_type=jnp.float32,
                     precision=lax.Precision.HIGHEST) + b2_ref[...]
        mu = jnp.mean(z2, axis=0, keepdims=True)
        var = jnp.mean(z2 * z2, axis=0, keepdims=True) - mu * mu
        zn = (z2 - mu) * lax.rsqrt(var + 1e-5) * g_ref[...] + be_ref[...]
        o_ref[...] = jnp.maximum(zn, 0.0) + hv

    smem = pl.BlockSpec(memory_space=pltpu.SMEM)
    vmem = pl.BlockSpec(memory_space=pltpu.VMEM)
    return pl.pallas_call(
        body,
        out_shape=jax.ShapeDtypeStruct((n, d), jnp.float32),
        in_specs=[smem] + [vmem] * 8,
        out_specs=vmem,
    )(jnp.reshape(eps, (1,)), h, parts, w1, jnp.reshape(b1, (1, d)), w2,
      jnp.reshape(b2, (1, d)), jnp.reshape(gamma, (1, d)),
      jnp.reshape(beta, (1, d)))


def _tc_head(parts, w1, b1, w2, b2):
    g, d = parts.shape[1], parts.shape[2]
    d_out = w2.shape[1]

    def body(p_ref, w1_ref, b1_ref, w2_ref, b2_ref, o_ref):
        zp = p_ref[0] + p_ref[1]
        a = jnp.dot(zp, w1_ref[...], preferred_element_type=jnp.float32,
                    precision=lax.Precision.HIGHEST) + b1_ref[...]
        a = jnp.maximum(a, 0.0)
        o_ref[...] = jnp.dot(a, w2_ref[...], preferred_element_type=jnp.float32,
                             precision=lax.Precision.HIGHEST) + b2_ref[...]

    vmem = pl.BlockSpec(memory_space=pltpu.VMEM)
    return pl.pallas_call(
        body,
        out_shape=jax.ShapeDtypeStruct((g, d_out), jnp.float32),
        in_specs=[vmem] * 5,
        out_specs=vmem,
    )(parts, w1, jnp.reshape(b1, (1, d)), w2, jnp.reshape(b2, (1, d_out)))


def kernel(x, edge_index, batch, l0_eps, l0_W1, l0_b1, l0_W2, l0_b2, l0_gamma,
           l0_beta, l1_eps, l1_W1, l1_b1, l1_W2, l1_b2, l1_gamma, l1_beta,
           l2_eps, l2_W1, l2_b1, l2_W2, l2_b2, l2_gamma, l2_beta, head_W1,
           head_b1, head_W2, head_b2):
    src = edge_index[0]
    dst = edge_index[1]
    n = x.shape[0]
    g = 512

    layers = [
        (l0_eps, l0_W1, l0_b1, l0_W2, l0_b2, l0_gamma, l0_beta),
        (l1_eps, l1_W1, l1_b1, l1_W2, l1_b2, l1_gamma, l1_beta),
        (l2_eps, l2_W1, l2_b1, l2_W2, l2_b2, l2_gamma, l2_beta),
    ]
    h = x
    for (eps, w1, b1, w2, b2, gamma, beta) in layers:
        parts = _sc_edge_segment_sum(h, src, dst, n, chunk=128)
        h = _tc_layer(h, parts, eps, w1, b1, w2, b2, gamma, beta)

    pool_parts = _sc_pool_segment_sum(h, batch, g, chunk=80)
    return _tc_head(pool_parts, head_W1, head_b1, head_W2, head_b2)


# speedup vs baseline: 3.2968x; 2.9869x over previous
import functools

import jax
import jax.numpy as jnp
from jax import lax
from jax.experimental import pallas as pl
from jax.experimental.pallas import tpu as pltpu
from jax.experimental.pallas import tpu_sc as plsc

NC = 2
NS = 16
NW = NC * NS


def _sc_edge_segment_sum(values, src, dst, num_segments, chunk=128):
    n_rows, d = values.shape
    n_edges = dst.shape[0]
    block = chunk * NW
    cpw = -(-n_edges // block)
    cpw = -(-cpw // 16) * 16
    e_pad = cpw * block
    pad = e_pad - n_edges
    n_dummy = 512
    if pad:
        src = jnp.concatenate(
            [src, jnp.arange(pad, dtype=jnp.int32) % n_rows])
        dst = jnp.concatenate(
            [dst, num_segments + (jnp.arange(pad, dtype=jnp.int32) % n_dummy)])
    idx4 = jnp.stack(
        [src.reshape(NW, cpw, chunk), dst.reshape(NW, cpw, chunk)], axis=2)
    seg_pad = num_segments + n_dummy
    rpt = (num_segments // NS) // 8 * 8
    tail = num_segments - rpt * NS
    zeros = jnp.zeros((seg_pad, d), jnp.float32)
    mesh = plsc.VectorSubcoreMesh(core_axis_name="c", subcore_axis_name="s")

    scratch = [
        pltpu.VMEM((2, 2, chunk), jnp.int32),
        pltpu.VMEM((2, chunk, d), jnp.float32),
        pltpu.VMEM_SHARED((seg_pad, d), jnp.float32),
        pltpu.SemaphoreType.DMA,
        pltpu.SemaphoreType.DMA,
        pltpu.SemaphoreType.DMA,
        pltpu.SemaphoreType.DMA,
    ]

    def body(val_hbm, idx_hbm, zero_hbm, out_hbm,
             idx_v, rows_v, acc, i0, i1, g0, g1):
        cid = lax.axis_index("c")
        sid = lax.axis_index("s")
        wid = sid * NC + cid
        isem = (i0, i1)
        gsem = (g0, g1)

        def idx_copy(c, b):
            return pltpu.make_async_copy(idx_hbm.at[wid, c], idx_v.at[b],
                                         isem[b])

        def gather(b):
            return pltpu.make_async_copy(val_hbm.at[idx_v.at[b, 0]],
                                         rows_v.at[b], gsem[b])

        def scatter(b):
            pltpu.sync_copy(rows_v.at[b], acc.at[idx_v.at[b, 1]], add=True)

        idx_copy(0, 0).start()
        idx_copy(1, 1).start()
        pltpu.sync_copy(zero_hbm.at[pl.ds(sid * rpt, rpt)],
                        acc.at[pl.ds(sid * rpt, rpt)])
        rest = seg_pad - rpt * NS
        @pl.when(sid == 0)
        def _():
            pltpu.sync_copy(zero_hbm.at[pl.ds(rpt * NS, rest)],
                            acc.at[pl.ds(rpt * NS, rest)])
        plsc.subcore_barrier()

        idx_copy(0, 0).wait()
        gather(0).start()

        def step(j, carry):
            for b in range(2):
                c = 2 * j + b
                nb = 1 - b
                idx_copy(c + 1, nb).wait()
                gather(nb).start()
                gather(b).wait()
                scatter(b)
                idx_copy(c + 2, b).start()
            return carry

        lax.fori_loop(0, (cpw - 2) // 2, step, 0)
        idx_copy(cpw - 1, 1).wait()
        gather(1).start()
        gather(0).wait()
        scatter(0)
        gather(1).wait()
        scatter(1)
        plsc.subcore_barrier()
        pltpu.sync_copy(acc.at[pl.ds(sid * rpt, rpt)],
                        out_hbm.at[cid, pl.ds(sid * rpt, rpt)])
        if tail:
            @pl.when(sid == 0)
            def _():
                pltpu.sync_copy(acc.at[pl.ds(rpt * NS, tail)],
                                out_hbm.at[cid, pl.ds(rpt * NS, tail)])

    return pl.kernel(
        body,
        out_type=jax.ShapeDtypeStruct((NC, num_segments, d), jnp.float32),
        mesh=mesh,
        scratch_types=scratch,
    )(values, idx4, zeros)


def _sc_pool_segment_sum(values, dst, num_segments, chunk=80):
    n_rows, d = values.shape
    n_chunks = n_rows // chunk
    rpt = (num_segments // NS) // 8 * 8
    tail = num_segments - rpt * NS
    zeros = jnp.zeros((num_segments, d), jnp.float32)
    mesh = plsc.VectorSubcoreMesh(core_axis_name="c", subcore_axis_name="s")

    scratch = [
        pltpu.VMEM((1, chunk), jnp.int32),
        pltpu.VMEM((chunk, d), jnp.float32),
        pltpu.VMEM_SHARED((num_segments, d), jnp.float32),
        pltpu.SemaphoreType.DMA,
    ]

    def body(val_hbm, dst_hbm, zero_hbm, out_hbm, dst_v, rows_v, acc, sem):
        cid = lax.axis_index("c")
        sid = lax.axis_index("s")
        wid = sid * NC + cid

        pltpu.sync_copy(zero_hbm.at[pl.ds(sid * rpt, rpt)],
                        acc.at[pl.ds(sid * rpt, rpt)])
        if tail:
            @pl.when(sid == 0)
            def _():
                pltpu.sync_copy(zero_hbm.at[pl.ds(rpt * NS, tail)],
                                acc.at[pl.ds(rpt * NS, tail)])
        plsc.subcore_barrier()

        n_my = (n_chunks - wid + NW - 1) // NW

        def step(i, carry):
            c = wid + i * NW
            off = c * chunk
            pltpu.sync_copy(dst_hbm.at[pl.ds(off, chunk)], dst_v.at[0])
            pltpu.sync_copy(val_hbm.at[pl.ds(off, chunk)], rows_v)
            pltpu.sync_copy(rows_v, acc.at[dst_v.at[0]], add=True)
            return carry

        lax.fori_loop(0, n_my, step, 0)
        plsc.subcore_barrier()
        pltpu.sync_copy(acc.at[pl.ds(sid * rpt, rpt)],
                        out_hbm.at[cid, pl.ds(sid * rpt, rpt)])
        if tail:
            @pl.when(sid == 0)
            def _():
                pltpu.sync_copy(acc.at[pl.ds(rpt * NS, tail)],
                                out_hbm.at[cid, pl.ds(rpt * NS, tail)])

    return pl.kernel(
        body,
        out_type=jax.ShapeDtypeStruct((NC, num_segments, d), jnp.float32),
        mesh=mesh,
        scratch_types=scratch,
    )(values, dst, zeros)


def _tc_layer(h, parts, eps, w1, b1, w2, b2, gamma, beta):
    n, d = h.shape

    def body(eps_ref, h_ref, p_ref, w1_ref, b1_ref, w2_ref, b2_ref,
             g_ref, be_ref, o_ref):
        hv = h_ref[...]
        z = hv + eps_ref[0] * hv + p_ref[0] + p_ref[1]
        a = jnp.dot(z, w1_ref[...], preferred_element_type=jnp.float32,
                    precision=lax.Precision.HIGHEST) + b1_ref[...]
        a = jnp.maximum(a, 0.0)
        z2 = jnp.dot(a, w2_ref[...], preferred_element_type=jnp.float32,
                     precision=lax.Precision.HIGHEST) + b2_ref[...]
        mu = jnp.mean(z2, axis=0, keepdims=True)
        var = jnp.mean(z2 * z2, axis=0, keepdims=True) - mu * mu
        zn = (z2 - mu) * lax.rsqrt(var + 1e-5) * g_ref[...] + be_ref[...]
        o_ref[...] = jnp.maximum(zn, 0.0) + hv

    smem = pl.BlockSpec(memory_space=pltpu.SMEM)
    vmem = pl.BlockSpec(memory_space=pltpu.VMEM)
    return pl.pallas_call(
        body,
        out_shape=jax.ShapeDtypeStruct((n, d), jnp.float32),
        in_specs=[smem] + [vmem] * 8,
        out_specs=vmem,
    )(jnp.reshape(eps, (1,)), h, parts, w1, jnp.reshape(b1, (1, d)), w2,
      jnp.reshape(b2, (1, d)), jnp.reshape(gamma, (1, d)),
      jnp.reshape(beta, (1, d)))


def _tc_head(parts, w1, b1, w2, b2):
    g, d = parts.shape[1], parts.shape[2]
    d_out = w2.shape[1]

    def body(p_ref, w1_ref, b1_ref, w2_ref, b2_ref, o_ref):
        zp = p_ref[0] + p_ref[1]
        a = jnp.dot(zp, w1_ref[...], preferred_element_type=jnp.float32,
                    precision=lax.Precision.HIGHEST) + b1_ref[...]
        a = jnp.maximum(a, 0.0)
        o_ref[...] = jnp.dot(a, w2_ref[...], preferred_element_type=jnp.float32,
                             precision=lax.Precision.HIGHEST) + b2_ref[...]

    vmem = pl.BlockSpec(memory_space=pltpu.VMEM)
    return pl.pallas_call(
        body,
        out_shape=jax.ShapeDtypeStruct((g, d_out), jnp.float32),
        in_specs=[vmem] * 5,
        out_specs=vmem,
    )(parts, w1, jnp.reshape(b1, (1, d)), w2, jnp.reshape(b2, (1, d_out)))


def kernel(x, edge_index, batch, l0_eps, l0_W1, l0_b1, l0_W2, l0_b2, l0_gamma,
           l0_beta, l1_eps, l1_W1, l1_b1, l1_W2, l1_b2, l1_gamma, l1_beta,
           l2_eps, l2_W1, l2_b1, l2_W2, l2_b2, l2_gamma, l2_beta, head_W1,
           head_b1, head_W2, head_b2):
    src = edge_index[0]
    dst = edge_index[1]
    n = x.shape[0]
    g = 512

    layers = [
        (l0_eps, l0_W1, l0_b1, l0_W2, l0_b2, l0_gamma, l0_beta),
        (l1_eps, l1_W1, l1_b1, l1_W2, l1_b2, l1_gamma, l1_beta),
        (l2_eps, l2_W1, l2_b1, l2_W2, l2_b2, l2_gamma, l2_beta),
    ]
    h = x
    for (eps, w1, b1, w2, b2, gamma, beta) in layers:
        parts = _sc_edge_segment_sum(h, src, dst, n, chunk=128)
        h = _tc_layer(h, parts, eps, w1, b1, w2, b2, gamma, beta)

    pool_parts = _sc_pool_segment_sum(h, batch, g, chunk=80)
    return _tc_head(pool_parts, head_W1, head_b1, head_W2, head_b2)


# speedup vs baseline: 3.6561x; 1.1090x over previous
import functools

import jax
import jax.numpy as jnp
from jax import lax
from jax.experimental import pallas as pl
from jax.experimental.pallas import tpu as pltpu
from jax.experimental.pallas import tpu_sc as plsc

NC = 2
NS = 16
NW = NC * NS


def _sc_edge_segment_sum(values, src, dst, num_segments, chunk=128):
    n_rows, d = values.shape
    n_edges = dst.shape[0]
    block = chunk * NW
    cpw = -(-n_edges // block)
    cpw = -(-cpw // 16) * 16
    e_pad = cpw * block
    pad = e_pad - n_edges
    n_dummy = 512
    if pad:
        src = jnp.concatenate(
            [src, jnp.arange(pad, dtype=jnp.int32) % n_rows])
        dst = jnp.concatenate(
            [dst, num_segments + (jnp.arange(pad, dtype=jnp.int32) % n_dummy)])
    idx4 = jnp.stack(
        [src.reshape(NW, cpw, chunk), dst.reshape(NW, cpw, chunk)], axis=2)
    seg_pad = num_segments + n_dummy
    rpt = (num_segments // NS) // 8 * 8
    tail = num_segments - rpt * NS
    zeros = jnp.zeros((seg_pad, d), jnp.float32)
    mesh = plsc.VectorSubcoreMesh(core_axis_name="c", subcore_axis_name="s")

    NI = 8
    scratch = [
        pltpu.VMEM((NI, 2, chunk), jnp.int32),
        pltpu.VMEM((2, chunk, d), jnp.float32),
        pltpu.VMEM_SHARED((seg_pad, d), jnp.float32),
    ] + [pltpu.SemaphoreType.DMA] * (NI + 4)

    def body(val_hbm, idx_hbm, zero_hbm, out_hbm, idx_v, rows_v, acc, *sems):
        cid = lax.axis_index("c")
        sid = lax.axis_index("s")
        wid = sid * NC + cid
        isem = sems[:NI]
        gsem = sems[NI:NI + 2]
        ssem = sems[NI + 2:]

        def idx_copy(c, s):
            return pltpu.make_async_copy(idx_hbm.at[wid, c], idx_v.at[s],
                                         isem[s])

        def gather(b, s):
            return pltpu.make_async_copy(val_hbm.at[idx_v.at[s, 0]],
                                         rows_v.at[b], gsem[b])

        def scatter_start(b, s):
            pltpu.async_copy(rows_v.at[b], acc.at[idx_v.at[s, 1]],
                             ssem[b], add=True)

        def scatter_wait(b, s):
            pltpu.make_async_copy(rows_v.at[b], acc.at[idx_v.at[s, 1]],
                                  ssem[b]).wait()

        for t in range(4):
            idx_copy(t, t).start()
        pltpu.sync_copy(zero_hbm.at[pl.ds(sid * rpt, rpt)],
                        acc.at[pl.ds(sid * rpt, rpt)])
        rest = seg_pad - rpt * NS
        @pl.when(sid == 0)
        def _():
            pltpu.sync_copy(zero_hbm.at[pl.ds(rpt * NS, rest)],
                            acc.at[pl.ds(rpt * NS, rest)])
        plsc.subcore_barrier()

        idx_copy(0, 0).wait()
        gather(0, 0).start()

        def step(j, carry):
            for k in range(NI):
                c = NI * j + k
                b = k % 2
                nb = 1 - b
                s = k
                ns = (k + 1) % NI
                @pl.when(c + 1 < cpw)
                def _():
                    idx_copy(c + 1, ns).wait()
                @pl.when(c >= 1)
                def _():
                    scatter_wait(nb, (k - 1) % NI)
                @pl.when(c + 1 < cpw)
                def _():
                    gather(nb, ns).start()
                gather(b, s).wait()
                scatter_start(b, s)
                @pl.when(c + 4 < cpw)
                def _():
                    idx_copy(c + 4, (k + 4) % NI).start()
            return carry

        lax.fori_loop(0, cpw // NI, step, 0)
        scatter_wait((cpw - 1) % 2, (cpw - 1) % NI)
        plsc.subcore_barrier()
        pltpu.sync_copy(acc.at[pl.ds(sid * rpt, rpt)],
                        out_hbm.at[cid, pl.ds(sid * rpt, rpt)])
        if tail:
            @pl.when(sid == 0)
            def _():
                pltpu.sync_copy(acc.at[pl.ds(rpt * NS, tail)],
                                out_hbm.at[cid, pl.ds(rpt * NS, tail)])

    return pl.kernel(
        body,
        out_type=jax.ShapeDtypeStruct((NC, num_segments, d), jnp.float32),
        mesh=mesh,
        scratch_types=scratch,
    )(values, idx4, zeros)


def _sc_pool_segment_sum(values, dst, num_segments, chunk=80):
    n_rows, d = values.shape
    n_chunks = n_rows // chunk
    rpt = (num_segments // NS) // 8 * 8
    tail = num_segments - rpt * NS
    zeros = jnp.zeros((num_segments, d), jnp.float32)
    mesh = plsc.VectorSubcoreMesh(core_axis_name="c", subcore_axis_name="s")

    scratch = [
        pltpu.VMEM((1, chunk), jnp.int32),
        pltpu.VMEM((chunk, d), jnp.float32),
        pltpu.VMEM_SHARED((num_segments, d), jnp.float32),
        pltpu.SemaphoreType.DMA,
    ]

    def body(val_hbm, dst_hbm, zero_hbm, out_hbm, dst_v, rows_v, acc, sem):
        cid = lax.axis_index("c")
        sid = lax.axis_index("s")
        wid = sid * NC + cid

        pltpu.sync_copy(zero_hbm.at[pl.ds(sid * rpt, rpt)],
                        acc.at[pl.ds(sid * rpt, rpt)])
        if tail:
            @pl.when(sid == 0)
            def _():
                pltpu.sync_copy(zero_hbm.at[pl.ds(rpt * NS, tail)],
                                acc.at[pl.ds(rpt * NS, tail)])
        plsc.subcore_barrier()

        n_my = (n_chunks - wid + NW - 1) // NW

        def step(i, carry):
            c = wid + i * NW
            off = c * chunk
            pltpu.sync_copy(dst_hbm.at[pl.ds(off, chunk)], dst_v.at[0])
            pltpu.sync_copy(val_hbm.at[pl.ds(off, chunk)], rows_v)
            pltpu.sync_copy(rows_v, acc.at[dst_v.at[0]], add=True)
            return carry

        lax.fori_loop(0, n_my, step, 0)
        plsc.subcore_barrier()
        pltpu.sync_copy(acc.at[pl.ds(sid * rpt, rpt)],
                        out_hbm.at[cid, pl.ds(sid * rpt, rpt)])
        if tail:
            @pl.when(sid == 0)
            def _():
                pltpu.sync_copy(acc.at[pl.ds(rpt * NS, tail)],
                                out_hbm.at[cid, pl.ds(rpt * NS, tail)])

    return pl.kernel(
        body,
        out_type=jax.ShapeDtypeStruct((NC, num_segments, d), jnp.float32),
        mesh=mesh,
        scratch_types=scratch,
    )(values, dst, zeros)


def _tc_layer(h, parts, eps, w1, b1, w2, b2, gamma, beta):
    n, d = h.shape

    def body(eps_ref, h_ref, p_ref, w1_ref, b1_ref, w2_ref, b2_ref,
             g_ref, be_ref, o_ref):
        hv = h_ref[...]
        z = hv + eps_ref[0] * hv + p_ref[0] + p_ref[1]
        a = jnp.dot(z, w1_ref[...], preferred_element_type=jnp.float32,
                    precision=lax.Precision.HIGHEST) + b1_ref[...]
        a = jnp.maximum(a, 0.0)
        z2 = jnp.dot(a, w2_ref[...], preferred_element_type=jnp.float32,
                     precision=lax.Precision.HIGHEST) + b2_ref[...]
        mu = jnp.mean(z2, axis=0, keepdims=True)
        var = jnp.mean(z2 * z2, axis=0, keepdims=True) - mu * mu
        zn = (z2 - mu) * lax.rsqrt(var + 1e-5) * g_ref[...] + be_ref[...]
        o_ref[...] = jnp.maximum(zn, 0.0) + hv

    smem = pl.BlockSpec(memory_space=pltpu.SMEM)
    vmem = pl.BlockSpec(memory_space=pltpu.VMEM)
    return pl.pallas_call(
        body,
        out_shape=jax.ShapeDtypeStruct((n, d), jnp.float32),
        in_specs=[smem] + [vmem] * 8,
        out_specs=vmem,
    )(jnp.reshape(eps, (1,)), h, parts, w1, jnp.reshape(b1, (1, d)), w2,
      jnp.reshape(b2, (1, d)), jnp.reshape(gamma, (1, d)),
      jnp.reshape(beta, (1, d)))


def _tc_head(parts, w1, b1, w2, b2):
    g, d = parts.shape[1], parts.shape[2]
    d_out = w2.shape[1]

    def body(p_ref, w1_ref, b1_ref, w2_ref, b2_ref, o_ref):
        zp = p_ref[0] + p_ref[1]
        a = jnp.dot(zp, w1_ref[...], preferred_element_type=jnp.float32,
                    precision=lax.Precision.HIGHEST) + b1_ref[...]
        a = jnp.maximum(a, 0.0)
        o_ref[...] = jnp.dot(a, w2_ref[...], preferred_element_type=jnp.float32,
                             precision=lax.Precision.HIGHEST) + b2_ref[...]

    vmem = pl.BlockSpec(memory_space=pltpu.VMEM)
    return pl.pallas_call(
        body,
        out_shape=jax.ShapeDtypeStruct((g, d_out), jnp.float32),
        in_specs=[vmem] * 5,
        out_specs=vmem,
    )(parts, w1, jnp.reshape(b1, (1, d)), w2, jnp.reshape(b2, (1, d_out)))


def kernel(x, edge_index, batch, l0_eps, l0_W1, l0_b1, l0_W2, l0_b2, l0_gamma,
           l0_beta, l1_eps, l1_W1, l1_b1, l1_W2, l1_b2, l1_gamma, l1_beta,
           l2_eps, l2_W1, l2_b1, l2_W2, l2_b2, l2_gamma, l2_beta, head_W1,
           head_b1, head_W2, head_b2):
    src = edge_index[0]
    dst = edge_index[1]
    n = x.shape[0]
    g = 512

    layers = [
        (l0_eps, l0_W1, l0_b1, l0_W2, l0_b2, l0_gamma, l0_beta),
        (l1_eps, l1_W1, l1_b1, l1_W2, l1_b2, l1_gamma, l1_beta),
        (l2_eps, l2_W1, l2_b1, l2_W2, l2_b2, l2_gamma, l2_beta),
    ]
    h = x
    for (eps, w1, b1, w2, b2, gamma, beta) in layers:
        parts = _sc_edge_segment_sum(h, src, dst, n, chunk=128)
        h = _tc_layer(h, parts, eps, w1, b1, w2, b2, gamma, beta)

    pool_parts = _sc_pool_segment_sum(h, batch, g, chunk=80)
    return _tc_head(pool_parts, head_W1, head_b1, head_W2, head_b2)
